# baseline (speedup 1.0000x reference)
import functools

import jax
import jax.numpy as jnp
from jax import lax
from jax.experimental import pallas as pl
from jax.experimental.pallas import tpu as pltpu
from jax.experimental.pallas import tpu_sc as plsc


def _relu(x):
    return jnp.maximum(x, 0.0)


def _gc_two_layers(support, u_feat, v_feat, W1, b1, W2, b2,
                   u_side_t, v_side_t, Wu1, bu1, Wv1, bv1, Wu2, Wv2, P):
    C, U, I = support.shape
    F = u_feat.shape[1]
    H1 = W1.shape[2]
    H2 = W2.shape[2]
    E = Wu1.shape[1]
    Ho = Wu2.shape[1]
    SB = 200
    NUB = U // SB
    KST = 3
    b1r = b1.reshape(1, H1)
    b2r = b2.reshape(1, H2)

    def body(s_ref, uf_ref, vf_ref, w1_ref, w2_ref,
             b1_ref, b2_ref, ust_ref, vst_ref, wu1_ref, bu1_ref,
             wv1_ref, bv1_ref, wu2_ref, wv2_ref, p_ref,
             ua_ref, vh2_ref,
             sbuf_ref, zu_ref, zv_ref, zu2_ref, zv2_ref,
             accv1_ref, accv2_ref):
        p = pl.program_id(0)
        i = pl.program_id(1)
        r = pl.program_id(2)
        first = (i == 0) & (r == 0)
        last = (i == NUB - 1) & (r == C - 1)
        row = i * SB

        def accum(sb, tv, tu, zdst_ref, accv_ref):
            cu = jnp.dot(sb, tv, preferred_element_type=jnp.float32)
            cvt = lax.dot_general(tu, sb, (((0,), (0,)), ((), ())),
                                  preferred_element_type=jnp.float32)

            @pl.when(r == 0)
            def _():
                zdst_ref[pl.ds(row, SB), :] = cu

            @pl.when(r > 0)
            def _():
                zdst_ref[pl.ds(row, SB), :] += cu

            @pl.when(first)
            def _():
                accv_ref[...] = cvt

            @pl.when(~first)
            def _():
                accv_ref[...] += cvt

        @pl.when(p == 0)
        def _():
            w = w1_ref[0]
            tv = lax.dot_general(vf_ref[...], w, (((1,), (1,)), ((), ())),
                                 preferred_element_type=jnp.float32
                                 ).astype(jnp.bfloat16)
            tu = lax.dot_general(uf_ref[...], w, (((1,), (1,)), ((), ())),
                                 preferred_element_type=jnp.float32
                                 ).astype(jnp.bfloat16)
            sb = s_ref[0].astype(jnp.bfloat16)

            @pl.when(r < KST)
            def _():
                sbuf_ref[pl.ds(r, 1), pl.ds(row, SB)] = sb[None]

            accum(sb, tv, tu, zu_ref, accv1_ref)

            @pl.when(last)
            def _():
                zu_ref[...] = _relu(zu_ref[...] + b1_ref[...])
                zv_ref[...] = _relu(accv1_ref[...].T + b1_ref[...])

        @pl.when(p == 1)
        def _():
            w = w2_ref[0]
            tv = lax.dot_general(zv_ref[...], w, (((1,), (1,)), ((), ())),
                                 preferred_element_type=jnp.float32
                                 ).astype(jnp.bfloat16)
            tu = lax.dot_general(zu_ref[pl.ds(row, SB), :], w,
                                 (((1,), (1,)), ((), ())),
                                 preferred_element_type=jnp.float32
                                 ).astype(jnp.bfloat16)

            @pl.when(r < KST)
            def _():
                accum(sbuf_ref[pl.ds(r, 1), pl.ds(row, SB)][0],
                      tv, tu, zu2_ref, accv2_ref)

            @pl.when(r >= KST)
            def _():
                accum(s_ref[0].astype(jnp.bfloat16),
                      tv, tu, zu2_ref, accv2_ref)

            @pl.when(last)
            def _():
                f32 = jnp.float32
                zu2 = _relu(zu2_ref[...] + b2_ref[...])
                zv2 = _relu(accv2_ref[...].T + b2_ref[...])
                uf2 = _relu(lax.dot_general(
                    ust_ref[...], wu1_ref[...], (((0,), (1,)), ((), ())),
                    preferred_element_type=f32) + bu1_ref[...])
                vf2 = _relu(lax.dot_general(
                    vst_ref[...], wv1_ref[...], (((0,), (1,)), ((), ())),
                    preferred_element_type=f32) + bv1_ref[...])
                uh = (jnp.dot(zu2, wu2_ref[0:H2, :],
                              preferred_element_type=f32)
                      + jnp.dot(uf2, wu2_ref[H2:H2 + E, :],
                                preferred_element_type=f32))
                vh = (jnp.dot(zv2, wv2_ref[0:H2, :],
                              preferred_element_type=f32)
                      + jnp.dot(vf2, wv2_ref[H2:H2 + E, :],
                                preferred_element_type=f32))
                ua_ref[...] = jnp.concatenate(
                    [jnp.dot(uh, p_ref[0], preferred_element_type=f32),
                     jnp.dot(uh, p_ref[1], preferred_element_type=f32)],
                    axis=1)
                vh2_ref[...] = jnp.concatenate([vh, vh], axis=1)

    def s_map(p, i, r):
        return (jnp.where(p == 0, r, jnp.maximum(r, KST)), i, 0)

    return pl.pallas_call(
        body,
        grid=(2, NUB, C),
        in_specs=[
            pl.BlockSpec((1, SB, I), s_map),
            pl.BlockSpec((SB, F), lambda p, i, r: (i, 0)),
            pl.BlockSpec((I, F), lambda p, i, r: (0, 0)),
            pl.BlockSpec((1, H1, F), lambda p, i, r: (r, 0, 0)),
            pl.BlockSpec((1, H2, H1), lambda p, i, r: (r, 0, 0)),
            pl.BlockSpec((1, H1), lambda p, i, r: (0, 0)),
            pl.BlockSpec((1, H2), lambda p, i, r: (0, 0)),
            pl.BlockSpec((E * 2, U), lambda p, i, r: (0, 0)),
            pl.BlockSpec((E * 2, I), lambda p, i, r: (0, 0)),
            pl.BlockSpec((E, E * 2), lambda p, i, r: (0, 0)),
            pl.BlockSpec((1, E), lambda p, i, r: (0, 0)),
            pl.BlockSpec((E, E * 2), lambda p, i, r: (0, 0)),
            pl.BlockSpec((1, E), lambda p, i, r: (0, 0)),
            pl.BlockSpec((H2 + E, Ho), lambda p, i, r: (0, 0)),
            pl.BlockSpec((H2 + E, Ho), lambda p, i, r: (0, 0)),
            pl.BlockSpec((2, Ho, Ho), lambda p, i, r: (0, 0, 0)),
        ],
        out_specs=[
            pl.BlockSpec((U, 2 * Ho), lambda p, i, r: (0, 0)),
            pl.BlockSpec((I, 2 * Ho), lambda p, i, r: (0, 0)),
        ],
        out_shape=[
            jax.ShapeDtypeStruct((U, 2 * Ho), jnp.float32),
            jax.ShapeDtypeStruct((I, 2 * Ho), jnp.float32),
        ],
        scratch_shapes=[
            pltpu.VMEM((KST, U, I), jnp.bfloat16),
            pltpu.VMEM((U, H1), jnp.float32),
            pltpu.VMEM((I, H1), jnp.float32),
            pltpu.VMEM((U, H2), jnp.float32),
            pltpu.VMEM((I, H2), jnp.float32),
            pltpu.VMEM((H1, I), jnp.float32),
            pltpu.VMEM((H2, I), jnp.float32),
        ],
        compiler_params=pltpu.CompilerParams(
            dimension_semantics=("arbitrary", "arbitrary", "arbitrary")),
    )(support, u_feat, v_feat,
      W1.transpose(0, 2, 1), W2.transpose(0, 2, 1), b1r, b2r,
      u_side_t, v_side_t, Wu1.T, bu1.reshape(1, E), Wv1.T,
      bv1.reshape(1, E), Wu2, Wv2, P)


def _pair_gather(uh, vh, u_idx, v_idx):
    B = u_idx.shape[0]
    D = uh.shape[1]
    NW = 32
    CH = 128
    PW = B // NW
    K = PW // CH
    mesh = plsc.VectorSubcoreMesh(core_axis_name="c", subcore_axis_name="s")

    @functools.partial(
        pl.kernel, mesh=mesh,
        out_type=[jax.ShapeDtypeStruct((B, D), jnp.float32),
                  jax.ShapeDtypeStruct((B, D), jnp.float32)],
        scratch_types=[
            pltpu.VMEM((K, CH), jnp.int32),
            pltpu.VMEM((K, CH), jnp.int32),
            pltpu.VMEM((PW, D), jnp.float32),
            pltpu.VMEM((PW, D), jnp.float32),
            pltpu.SemaphoreType.DMA,
        ],
    )
    def k(uh_hbm, u_hbm, vh_hbm, v_hbm, out_u, out_v,
          uix, vix, urows, vrows, sem):
        wid = lax.axis_index("s") * 2 + lax.axis_index("c")
        base = wid * PW
        for j in range(K):
            pltpu.sync_copy(u_hbm.at[pl.ds(base + j * CH, CH)], uix.at[j])
            pltpu.sync_copy(v_hbm.at[pl.ds(base + j * CH, CH)], vix.at[j])
        copies = []
        for j in range(K):
            sl = pl.ds(j * CH, CH)
            copies.append(pltpu.async_copy(
                uh_hbm.at[uix.at[j]], urows.at[sl], sem))
            copies.append(pltpu.async_copy(
                vh_hbm.at[vix.at[j]], vrows.at[sl], sem))
        for c in copies:
            c.wait()
        pltpu.sync_copy(urows, out_u.at[pl.ds(base, PW)])
        pltpu.sync_copy(vrows, out_v.at[pl.ds(base, PW)])

    return k(uh, u_idx, vh, v_idx)


def _decode(U_g, V_g, a_comb):
    B, D2 = U_g.shape
    D = D2 // 2
    NCLS = a_comb.shape[1]
    NBLK = 4
    BB = B // NBLK

    def body(u_ref, v_ref, a_ref, out_ref, mh_ref):
        f32 = jnp.float32
        a = a_ref[...]
        mm = jnp.concatenate(
            [jnp.broadcast_to(a[0:1, :], (D, NCLS)),
             jnp.broadcast_to(a[1:2, :], (D, NCLS))], axis=0)
        prod = (u_ref[...] * v_ref[...]).astype(f32)
        logits = jnp.dot(prod, mm, preferred_element_type=f32)
        m = jnp.max(logits, axis=1, keepdims=True)
        e = jnp.exp(logits - m)
        sm = e / jnp.sum(e, axis=1, keepdims=True)
        cls = (lax.broadcasted_iota(jnp.int32, (1, NCLS), 1) + 1).astype(f32)
        out_ref[...] = logits.T
        mh_ref[...] = jnp.sum(sm * cls, axis=1)

    out_t, mh = pl.pallas_call(
        body,
        grid=(NBLK,),
        in_specs=[
            pl.BlockSpec((BB, D2), lambda i: (i, 0)),
            pl.BlockSpec((BB, D2), lambda i: (i, 0)),
            pl.BlockSpec((2, NCLS), lambda i: (0, 0)),
        ],
        out_specs=[
            pl.BlockSpec((NCLS, BB), lambda i: (0, i)),
            pl.BlockSpec((BB,), lambda i: (i,)),
        ],
        out_shape=[
            jax.ShapeDtypeStruct((NCLS, B), jnp.float32),
            jax.ShapeDtypeStruct((B,), jnp.float32),
        ],
    )(U_g, V_g, a_comb)
    return out_t.T, mh


def kernel(u, v, r_matrix, u_features, v_features, u_features_side,
           v_features_side, W1, b1, W2, b2, Wu1, bu1, Wv1, bv1, Wu2, Wv2,
           P, a_comb):
    ua, vh = _gc_two_layers(r_matrix, u_features, v_features, W1, b1, W2, b2,
                            u_features_side.T, v_features_side.T,
                            Wu1, bu1, Wv1, bv1, Wu2, Wv2, P)
    U_g, V_g = _pair_gather(ua, vh, u, v)
    return _decode(U_g, V_g, a_comb)


# speedup vs baseline: 1.4282x; 1.4282x over previous
import functools

import jax
import jax.numpy as jnp
from jax import lax
from jax.experimental import pallas as pl
from jax.experimental.pallas import tpu as pltpu
from jax.experimental.pallas import tpu_sc as plsc


def _relu(x):
    return jnp.maximum(x, 0.0)


def _gc_two_layers(support, u_feat, v_feat, W1, b1, W2, b2,
                   u_side_t, v_side_t, Wu1, bu1, Wv1, bv1, Wu2, Wv2, P):
    C, U, I = support.shape
    F = u_feat.shape[1]
    H1 = W1.shape[2]
    H2 = W2.shape[2]
    E = Wu1.shape[1]
    Ho = Wu2.shape[1]
    SB = 600
    NUB = U // SB
    KST = 2
    b1r = b1.reshape(1, H1)
    b2r = b2.reshape(1, H2)

    def body(s_ref, uf_ref, vf_ref, w1_ref, w2_ref,
             b1_ref, b2_ref, ust_ref, vst_ref, wu1_ref, bu1_ref,
             wv1_ref, bv1_ref, wu2_ref, wv2_ref, p_ref,
             ua_ref, vh2_ref,
             sbuf_ref, zu_ref, zv_ref, zu2_ref, zv2_ref, accv1_ref):
        accv2_ref = accv1_ref.at[pl.ds(0, H2), :]
        p = pl.program_id(0)
        i = pl.program_id(1)
        r = pl.program_id(2)
        first = (i == 0) & (r == 0)
        last = (i == NUB - 1) & (r == C - 1)
        row = i * SB

        def accum(sb, tv, tu, zdst_ref, accv_ref):
            cu = jnp.dot(sb, tv, preferred_element_type=jnp.float32)
            cvt = lax.dot_general(tu, sb, (((0,), (0,)), ((), ())),
                                  preferred_element_type=jnp.float32)

            @pl.when(r == 0)
            def _():
                zdst_ref[pl.ds(row, SB), :] = cu

            @pl.when(r > 0)
            def _():
                zdst_ref[pl.ds(row, SB), :] += cu

            @pl.when(first)
            def _():
                accv_ref[...] = cvt

            @pl.when(~first)
            def _():
                accv_ref[...] += cvt

        @pl.when(p == 0)
        def _():
            w = w1_ref[0]
            tv = lax.dot_general(vf_ref[...], w, (((1,), (1,)), ((), ())),
                                 preferred_element_type=jnp.float32
                                 ).astype(jnp.bfloat16)
            tu = lax.dot_general(uf_ref[...], w, (((1,), (1,)), ((), ())),
                                 preferred_element_type=jnp.float32
                                 ).astype(jnp.bfloat16)
            sb = s_ref[0].astype(jnp.bfloat16)

            @pl.when(r < KST)
            def _():
                sbuf_ref[pl.ds(r, 1), pl.ds(row, SB)] = sb[None]

            accum(sb, tv, tu, zu_ref, accv1_ref)

            @pl.when(last)
            def _():
                zu_ref[...] = _relu(zu_ref[...] + b1_ref[...])
                zv_ref[...] = _relu(accv1_ref[...].T + b1_ref[...])

        @pl.when(p == 1)
        def _():
            w = w2_ref[0]
            tv = lax.dot_general(zv_ref[...], w, (((1,), (1,)), ((), ())),
                                 preferred_element_type=jnp.float32
                                 ).astype(jnp.bfloat16)
            tu = lax.dot_general(zu_ref[pl.ds(row, SB), :], w,
                                 (((1,), (1,)), ((), ())),
                                 preferred_element_type=jnp.float32
                                 ).astype(jnp.bfloat16)

            @pl.when(r < KST)
            def _():
                accum(sbuf_ref[pl.ds(r, 1), pl.ds(row, SB)][0],
                      tv, tu, zu2_ref, accv2_ref)

            @pl.when(r >= KST)
            def _():
                accum(s_ref[0].astype(jnp.bfloat16),
                      tv, tu, zu2_ref, accv2_ref)

            @pl.when(last)
            def _():
                f32 = jnp.float32
                zu2 = _relu(zu2_ref[...] + b2_ref[...])
                zv2 = _relu(accv2_ref[...].T + b2_ref[...])
                uf2 = _relu(lax.dot_general(
                    ust_ref[...], wu1_ref[...], (((0,), (1,)), ((), ())),
                    preferred_element_type=f32) + bu1_ref[...])
                vf2 = _relu(lax.dot_general(
                    vst_ref[...], wv1_ref[...], (((0,), (1,)), ((), ())),
                    preferred_element_type=f32) + bv1_ref[...])
                uh = (jnp.dot(zu2, wu2_ref[0:H2, :],
                              preferred_element_type=f32)
                      + jnp.dot(uf2, wu2_ref[H2:H2 + E, :],
                                preferred_element_type=f32))
                vh = (jnp.dot(zv2, wv2_ref[0:H2, :],
                              preferred_element_type=f32)
                      + jnp.dot(vf2, wv2_ref[H2:H2 + E, :],
                                preferred_element_type=f32))
                ua_ref[...] = jnp.concatenate(
                    [jnp.dot(uh, p_ref[0], preferred_element_type=f32),
                     jnp.dot(uh, p_ref[1], preferred_element_type=f32)],
                    axis=1)
                vh2_ref[...] = jnp.concatenate([vh, vh], axis=1)

    def s_map(p, i, r):
        return (jnp.where(p == 0, r, jnp.maximum(r, KST)), i, 0)

    return pl.pallas_call(
        body,
        grid=(2, NUB, C),
        in_specs=[
            pl.BlockSpec((1, SB, I), s_map),
            pl.BlockSpec((SB, F), lambda p, i, r: (i, 0)),
            pl.BlockSpec((I, F), lambda p, i, r: (0, 0)),
            pl.BlockSpec((1, H1, F), lambda p, i, r: (r, 0, 0)),
            pl.BlockSpec((1, H2, H1), lambda p, i, r: (r, 0, 0)),
            pl.BlockSpec((1, H1), lambda p, i, r: (0, 0)),
            pl.BlockSpec((1, H2), lambda p, i, r: (0, 0)),
            pl.BlockSpec((E * 2, U), lambda p, i, r: (0, 0)),
            pl.BlockSpec((E * 2, I), lambda p, i, r: (0, 0)),
            pl.BlockSpec((E, E * 2), lambda p, i, r: (0, 0)),
            pl.BlockSpec((1, E), lambda p, i, r: (0, 0)),
            pl.BlockSpec((E, E * 2), lambda p, i, r: (0, 0)),
            pl.BlockSpec((1, E), lambda p, i, r: (0, 0)),
            pl.BlockSpec((H2 + E, Ho), lambda p, i, r: (0, 0)),
            pl.BlockSpec((H2 + E, Ho), lambda p, i, r: (0, 0)),
            pl.BlockSpec((2, Ho, Ho), lambda p, i, r: (0, 0, 0)),
        ],
        out_specs=[
            pl.BlockSpec((U, 2 * Ho), lambda p, i, r: (0, 0)),
            pl.BlockSpec((I, 2 * Ho), lambda p, i, r: (0, 0)),
        ],
        out_shape=[
            jax.ShapeDtypeStruct((U, 2 * Ho), jnp.float32),
            jax.ShapeDtypeStruct((I, 2 * Ho), jnp.float32),
        ],
        scratch_shapes=[
            pltpu.VMEM((KST, U, I), jnp.bfloat16),
            pltpu.VMEM((U, H1), jnp.float32),
            pltpu.VMEM((I, H1), jnp.float32),
            pltpu.VMEM((U, H2), jnp.float32),
            pltpu.VMEM((I, H2), jnp.float32),
            pltpu.VMEM((H1, I), jnp.float32),
        ],
        compiler_params=pltpu.CompilerParams(
            dimension_semantics=("arbitrary", "arbitrary", "arbitrary")),
    )(support, u_feat, v_feat,
      W1.transpose(0, 2, 1), W2.transpose(0, 2, 1), b1r, b2r,
      u_side_t, v_side_t, Wu1.T, bu1.reshape(1, E), Wv1.T,
      bv1.reshape(1, E), Wu2, Wv2, P)


def _pair_gather(uh, vh, u_idx, v_idx):
    B = u_idx.shape[0]
    D = uh.shape[1]
    NW = 32
    CH = 128
    PW = B // NW
    K = PW // CH
    mesh = plsc.VectorSubcoreMesh(core_axis_name="c", subcore_axis_name="s")

    @functools.partial(
        pl.kernel, mesh=mesh,
        out_type=[jax.ShapeDtypeStruct((B, D), jnp.float32),
                  jax.ShapeDtypeStruct((B, D), jnp.float32)],
        scratch_types=[
            pltpu.VMEM((K, CH), jnp.int32),
            pltpu.VMEM((K, CH), jnp.int32),
            pltpu.VMEM((PW, D), jnp.float32),
            pltpu.VMEM((PW, D), jnp.float32),
            pltpu.SemaphoreType.DMA,
        ],
    )
    def k(uh_hbm, u_hbm, vh_hbm, v_hbm, out_u, out_v,
          uix, vix, urows, vrows, sem):
        wid = lax.axis_index("s") * 2 + lax.axis_index("c")
        base = wid * PW
        for j in range(K):
            pltpu.sync_copy(u_hbm.at[pl.ds(base + j * CH, CH)], uix.at[j])
            pltpu.sync_copy(v_hbm.at[pl.ds(base + j * CH, CH)], vix.at[j])
        copies = []
        for j in range(K):
            sl = pl.ds(j * CH, CH)
            copies.append(pltpu.async_copy(
                uh_hbm.at[uix.at[j]], urows.at[sl], sem))
            copies.append(pltpu.async_copy(
                vh_hbm.at[vix.at[j]], vrows.at[sl], sem))
        for c in copies:
            c.wait()
        pltpu.sync_copy(urows, out_u.at[pl.ds(base, PW)])
        pltpu.sync_copy(vrows, out_v.at[pl.ds(base, PW)])

    return k(uh, u_idx, vh, v_idx)


def _decode(U_g, V_g, a_comb):
    B, D2 = U_g.shape
    D = D2 // 2
    NCLS = a_comb.shape[1]
    NBLK = 4
    BB = B // NBLK

    def body(u_ref, v_ref, a_ref, out_ref, mh_ref):
        f32 = jnp.float32
        a = a_ref[...]
        mm = jnp.concatenate(
            [jnp.broadcast_to(a[0:1, :], (D, NCLS)),
             jnp.broadcast_to(a[1:2, :], (D, NCLS))], axis=0)
        prod = (u_ref[...] * v_ref[...]).astype(f32)
        logits = jnp.dot(prod, mm, preferred_element_type=f32)
        m = jnp.max(logits, axis=1, keepdims=True)
        e = jnp.exp(logits - m)
        sm = e / jnp.sum(e, axis=1, keepdims=True)
        cls = (lax.broadcasted_iota(jnp.int32, (1, NCLS), 1) + 1).astype(f32)
        out_ref[...] = logits.T
        mh_ref[...] = jnp.sum(sm * cls, axis=1)

    out_t, mh = pl.pallas_call(
        body,
        grid=(NBLK,),
        in_specs=[
            pl.BlockSpec((BB, D2), lambda i: (i, 0)),
            pl.BlockSpec((BB, D2), lambda i: (i, 0)),
            pl.BlockSpec((2, NCLS), lambda i: (0, 0)),
        ],
        out_specs=[
            pl.BlockSpec((NCLS, BB), lambda i: (0, i)),
            pl.BlockSpec((BB,), lambda i: (i,)),
        ],
        out_shape=[
            jax.ShapeDtypeStruct((NCLS, B), jnp.float32),
            jax.ShapeDtypeStruct((B,), jnp.float32),
        ],
    )(U_g, V_g, a_comb)
    return out_t.T, mh


def kernel(u, v, r_matrix, u_features, v_features, u_features_side,
           v_features_side, W1, b1, W2, b2, Wu1, bu1, Wv1, bv1, Wu2, Wv2,
           P, a_comb):
    ua, vh = _gc_two_layers(r_matrix, u_features, v_features, W1, b1, W2, b2,
                            u_features_side.T, v_features_side.T,
                            Wu1, bu1, Wv1, bv1, Wu2, Wv2, P)
    U_g, V_g = _pair_gather(ua, vh, u, v)
    return _decode(U_g, V_g, a_comb)


# speedup vs baseline: 1.4573x; 1.0204x over previous
import functools

import jax
import jax.numpy as jnp
from jax import lax
from jax.experimental import pallas as pl
from jax.experimental.pallas import tpu as pltpu
from jax.experimental.pallas import tpu_sc as plsc


def _relu(x):
    return jnp.maximum(x, 0.0)


def _gc_two_layers(support, u_feat, v_feat, W1, b1, W2, b2,
                   u_side_t, v_side_t, Wu1, bu1, Wv1, bv1, Wu2, Wv2, P):
    C, U, I = support.shape
    F = u_feat.shape[1]
    H1 = W1.shape[2]
    H2 = W2.shape[2]
    E = Wu1.shape[1]
    Ho = Wu2.shape[1]
    SB = 600
    NUB = U // SB
    KST = 2
    b1r = b1.reshape(1, H1)
    b2r = b2.reshape(1, H2)

    def body(s_ref, uf_ref, vf_ref, w1_ref, w2_ref,
             b1_ref, b2_ref, ust_ref, vst_ref, wu1_ref, bu1_ref,
             wv1_ref, bv1_ref, wu2_ref, wv2_ref, p_ref,
             ua_ref, vh2_ref,
             sbuf_ref, zu_ref, zv_ref, zu2_ref, zv2_ref, accv1_ref,
             tmpv1_ref, tmpv2_ref):
        accv2_ref = accv1_ref.at[pl.ds(0, H2), :]
        p = pl.program_id(0)
        i = pl.program_id(1)
        r = pl.program_id(2)
        first = (i == 0) & (r == 0)
        last = (i == NUB - 1) & (r == C - 1)
        row = i * SB

        def accum(sb, tv, tu, zdst_ref, accv_ref):
            cu = jnp.dot(sb, tv, preferred_element_type=jnp.float32)
            cvt = lax.dot_general(tu, sb, (((0,), (0,)), ((), ())),
                                  preferred_element_type=jnp.float32)

            @pl.when(r == 0)
            def _():
                zdst_ref[pl.ds(row, SB), :] = cu

            @pl.when(r > 0)
            def _():
                zdst_ref[pl.ds(row, SB), :] += cu

            @pl.when(first)
            def _():
                accv_ref[...] = cvt

            @pl.when(~first)
            def _():
                accv_ref[...] += cvt

        @pl.when(p == 0)
        def _():
            w = w1_ref[0].astype(jnp.bfloat16)

            @pl.when(i == 0)
            def _():
                tmpv1_ref[pl.ds(r, 1)] = lax.dot_general(
                    vf_ref[...].astype(jnp.bfloat16), w,
                    (((1,), (1,)), ((), ())),
                    preferred_element_type=jnp.float32
                ).astype(jnp.bfloat16)[None]

            tv = tmpv1_ref[pl.ds(r, 1)][0]
            tu = lax.dot_general(uf_ref[...].astype(jnp.bfloat16), w,
                                 (((1,), (1,)), ((), ())),
                                 preferred_element_type=jnp.float32
                                 ).astype(jnp.bfloat16)
            sb = s_ref[0].astype(jnp.bfloat16)

            @pl.when(r < KST)
            def _():
                sbuf_ref[pl.ds(r, 1), pl.ds(i, 1)] = sb[None, None]

            accum(sb, tv, tu, zu_ref, accv1_ref)

            @pl.when(last)
            def _():
                zu_ref[...] = _relu(zu_ref[...] + b1_ref[...])
                zv_ref[...] = _relu(accv1_ref[...].T + b1_ref[...])

        @pl.when(p == 1)
        def _():
            w = w2_ref[0].astype(jnp.bfloat16)

            @pl.when(i == 0)
            def _():
                tmpv2_ref[pl.ds(r, 1)] = lax.dot_general(
                    zv_ref[...].astype(jnp.bfloat16), w,
                    (((1,), (1,)), ((), ())),
                    preferred_element_type=jnp.float32
                ).astype(jnp.bfloat16)[None]

            tv = tmpv2_ref[pl.ds(r, 1)][0]
            tu = lax.dot_general(zu_ref[pl.ds(row, SB), :].astype(jnp.bfloat16),
                                 w, (((1,), (1,)), ((), ())),
                                 preferred_element_type=jnp.float32
                                 ).astype(jnp.bfloat16)

            @pl.when(r < KST)
            def _():
                accum(sbuf_ref[pl.ds(r, 1), pl.ds(i, 1)][0, 0],
                      tv, tu, zu2_ref, accv2_ref)

            @pl.when(r >= KST)
            def _():
                accum(s_ref[0].astype(jnp.bfloat16),
                      tv, tu, zu2_ref, accv2_ref)

            @pl.when(last)
            def _():
                f32 = jnp.float32
                bf = jnp.bfloat16
                zu2 = _relu(zu2_ref[...] + b2_ref[...]).astype(bf)
                zv2 = _relu(accv2_ref[...].T + b2_ref[...]).astype(bf)
                uf2 = _relu(lax.dot_general(
                    ust_ref[...].astype(bf), wu1_ref[...].astype(bf),
                    (((0,), (1,)), ((), ())),
                    preferred_element_type=f32) + bu1_ref[...]).astype(bf)
                vf2 = _relu(lax.dot_general(
                    vst_ref[...].astype(bf), wv1_ref[...].astype(bf),
                    (((0,), (1,)), ((), ())),
                    preferred_element_type=f32) + bv1_ref[...]).astype(bf)
                wu2 = wu2_ref[...].astype(bf)
                wv2 = wv2_ref[...].astype(bf)
                uh = (jnp.dot(zu2, wu2[0:H2, :], preferred_element_type=f32)
                      + jnp.dot(uf2, wu2[H2:H2 + E, :],
                                preferred_element_type=f32)).astype(bf)
                vh = (jnp.dot(zv2, wv2[0:H2, :], preferred_element_type=f32)
                      + jnp.dot(vf2, wv2[H2:H2 + E, :],
                                preferred_element_type=f32))
                ua_ref[...] = jnp.concatenate(
                    [jnp.dot(uh, p_ref[0].astype(bf),
                             preferred_element_type=f32),
                     jnp.dot(uh, p_ref[1].astype(bf),
                             preferred_element_type=f32)],
                    axis=1)
                vh2_ref[...] = jnp.concatenate([vh, vh], axis=1)

    def s_map(p, i, r):
        return (jnp.where(p == 0, r, jnp.maximum(r, KST)), i, 0)

    return pl.pallas_call(
        body,
        grid=(2, NUB, C),
        in_specs=[
            pl.BlockSpec((1, SB, I), s_map),
            pl.BlockSpec((SB, F), lambda p, i, r: (i, 0)),
            pl.BlockSpec((I, F), lambda p, i, r: (0, 0)),
            pl.BlockSpec((1, H1, F), lambda p, i, r: (r, 0, 0)),
            pl.BlockSpec((1, H2, H1), lambda p, i, r: (r, 0, 0)),
            pl.BlockSpec((1, H1), lambda p, i, r: (0, 0)),
            pl.BlockSpec((1, H2), lambda p, i, r: (0, 0)),
            pl.BlockSpec((E * 2, U), lambda p, i, r: (0, 0)),
            pl.BlockSpec((E * 2, I), lambda p, i, r: (0, 0)),
            pl.BlockSpec((E, E * 2), lambda p, i, r: (0, 0)),
            pl.BlockSpec((1, E), lambda p, i, r: (0, 0)),
            pl.BlockSpec((E, E * 2), lambda p, i, r: (0, 0)),
            pl.BlockSpec((1, E), lambda p, i, r: (0, 0)),
            pl.BlockSpec((H2 + E, Ho), lambda p, i, r: (0, 0)),
            pl.BlockSpec((H2 + E, Ho), lambda p, i, r: (0, 0)),
            pl.BlockSpec((2, Ho, Ho), lambda p, i, r: (0, 0, 0)),
        ],
        out_specs=[
            pl.BlockSpec((U, 2 * Ho), lambda p, i, r: (0, 0)),
            pl.BlockSpec((I, 2 * Ho), lambda p, i, r: (0, 0)),
        ],
        out_shape=[
            jax.ShapeDtypeStruct((U, 2 * Ho), jnp.float32),
            jax.ShapeDtypeStruct((I, 2 * Ho), jnp.float32),
        ],
        scratch_shapes=[
            pltpu.VMEM((KST, NUB, SB, I), jnp.bfloat16),
            pltpu.VMEM((U, H1), jnp.float32),
            pltpu.VMEM((I, H1), jnp.float32),
            pltpu.VMEM((U, H2), jnp.float32),
            pltpu.VMEM((I, H2), jnp.float32),
            pltpu.VMEM((H1, I), jnp.float32),
            pltpu.VMEM((C, I, H1), jnp.bfloat16),
            pltpu.VMEM((C, I, H2), jnp.bfloat16),
        ],
        compiler_params=pltpu.CompilerParams(
            dimension_semantics=("arbitrary", "arbitrary", "arbitrary")),
    )(support, u_feat, v_feat,
      W1.transpose(0, 2, 1), W2.transpose(0, 2, 1), b1r, b2r,
      u_side_t, v_side_t, Wu1.T, bu1.reshape(1, E), Wv1.T,
      bv1.reshape(1, E), Wu2, Wv2, P)


def _pair_gather(uh, vh, u_idx, v_idx):
    B = u_idx.shape[0]
    D = uh.shape[1]
    NW = 32
    CH = 128
    PW = B // NW
    K = PW // CH
    mesh = plsc.VectorSubcoreMesh(core_axis_name="c", subcore_axis_name="s")

    @functools.partial(
        pl.kernel, mesh=mesh,
        out_type=[jax.ShapeDtypeStruct((B, D), jnp.float32),
                  jax.ShapeDtypeStruct((B, D), jnp.float32)],
        scratch_types=[
            pltpu.VMEM((K, CH), jnp.int32),
            pltpu.VMEM((K, CH), jnp.int32),
            pltpu.VMEM((PW, D), jnp.float32),
            pltpu.VMEM((PW, D), jnp.float32),
            pltpu.SemaphoreType.DMA,
        ],
    )
    def k(uh_hbm, u_hbm, vh_hbm, v_hbm, out_u, out_v,
          uix, vix, urows, vrows, sem):
        wid = lax.axis_index("s") * 2 + lax.axis_index("c")
        base = wid * PW
        for j in range(K):
            pltpu.sync_copy(u_hbm.at[pl.ds(base + j * CH, CH)], uix.at[j])
            pltpu.sync_copy(v_hbm.at[pl.ds(base + j * CH, CH)], vix.at[j])
        copies = []
        for j in range(K):
            sl = pl.ds(j * CH, CH)
            copies.append(pltpu.async_copy(
                uh_hbm.at[uix.at[j]], urows.at[sl], sem))
            copies.append(pltpu.async_copy(
                vh_hbm.at[vix.at[j]], vrows.at[sl], sem))
        for c in copies:
            c.wait()
        pltpu.sync_copy(urows, out_u.at[pl.ds(base, PW)])
        pltpu.sync_copy(vrows, out_v.at[pl.ds(base, PW)])

    return k(uh, u_idx, vh, v_idx)


def _decode(U_g, V_g, a_comb):
    B, D2 = U_g.shape
    D = D2 // 2
    NCLS = a_comb.shape[1]
    NBLK = 4
    BB = B // NBLK

    def body(u_ref, v_ref, a_ref, out_ref, mh_ref):
        f32 = jnp.float32
        a = a_ref[...].astype(jnp.bfloat16).astype(f32)
        prod = u_ref[...] * v_ref[...]
        b0 = jnp.sum(prod[:, 0:D], axis=1, keepdims=True).astype(
            jnp.bfloat16).astype(f32)
        b1 = jnp.sum(prod[:, D:2 * D], axis=1, keepdims=True).astype(
            jnp.bfloat16).astype(f32)
        logits = b0 * a[0:1, :] + b1 * a[1:2, :]
        m = jnp.max(logits, axis=1, keepdims=True)
        e = jnp.exp(logits - m)
        sm = e / jnp.sum(e, axis=1, keepdims=True)
        cls = (lax.broadcasted_iota(jnp.int32, (1, NCLS), 1) + 1).astype(f32)
        out_ref[...] = logits.T
        mh_ref[...] = jnp.sum(sm * cls, axis=1)

    out_t, mh = pl.pallas_call(
        body,
        grid=(NBLK,),
        in_specs=[
            pl.BlockSpec((BB, D2), lambda i: (i, 0)),
            pl.BlockSpec((BB, D2), lambda i: (i, 0)),
            pl.BlockSpec((2, NCLS), lambda i: (0, 0)),
        ],
        out_specs=[
            pl.BlockSpec((NCLS, BB), lambda i: (0, i)),
            pl.BlockSpec((BB,), lambda i: (i,)),
        ],
        out_shape=[
            jax.ShapeDtypeStruct((NCLS, B), jnp.float32),
            jax.ShapeDtypeStruct((B,), jnp.float32),
        ],
    )(U_g, V_g, a_comb)
    return out_t.T, mh


def kernel(u, v, r_matrix, u_features, v_features, u_features_side,
           v_features_side, W1, b1, W2, b2, Wu1, bu1, Wv1, bv1, Wu2, Wv2,
           P, a_comb):
    ua, vh = _gc_two_layers(r_matrix, u_features, v_features, W1, b1, W2, b2,
                            u_features_side.T, v_features_side.T,
                            Wu1, bu1, Wv1, bv1, Wu2, Wv2, P)
    U_g, V_g = _pair_gather(ua, vh, u, v)
    return _decode(U_g, V_g, a_comb)
